# Initial kernel scaffold; baseline (speedup 1.0000x reference)
import jax
import jax.numpy as jnp
from jax.experimental import pallas as pl


def kernel(resources, operations, requirement_edges, W_res, W_op, att_self_coef, att_coef):
    raise NotImplementedError("write your pallas kernel here")


# speedup vs baseline: 11.1231x; 11.1231x over previous
import jax
import jax.numpy as jnp
from jax import lax
from jax.experimental import pallas as pl
from jax.experimental.pallas import tpu as pltpu
import jax.experimental.pallas.tpu_sc as plsc

N = 10000
E = 320000
HID = 8

NC = 2
NS = 16
NW = NC * NS
COL_SPLIT = HID
EDGE_SPLIT = NW // COL_SPLIT
E_PER_W = E // EDGE_SPLIT
CHUNK = 8000
N_CHUNKS = E_PER_W // CHUNK
STEPS = CHUNK // 16
NF = N * HID


def _dense_body(res_in, ops_in, wres, wop, a_self, c_res, c_ops,
                p_out, ops_out, rs_out, os_out, sself_out):
    dn = (((1,), (1,)), ((), ()))
    res_t = lax.dot_general(wres[...].astype(jnp.bfloat16),
                            res_in[...].astype(jnp.bfloat16), dn,
                            preferred_element_type=jnp.float32)
    ops_t = lax.dot_general(wop[...].astype(jnp.bfloat16),
                            ops_in[...].astype(jnp.bfloat16), dn,
                            preferred_element_type=jnp.float32)
    ops_out[...] = ops_t
    res_tb = res_t.astype(jnp.bfloat16).astype(jnp.float32)
    ops_tb = ops_t.astype(jnp.bfloat16).astype(jnp.float32)
    rs_out[...] = jnp.sum(res_tb * c_res[...], axis=0, keepdims=True)
    os_out[...] = jnp.sum(ops_tb * c_ops[...], axis=0, keepdims=True)
    sl = jnp.sum(res_tb * a_self[...], axis=0, keepdims=True)
    sl = jnp.where(sl > 0, sl, sl * 0.2)
    self_num = jnp.exp(sl)
    p_out[...] = res_t * self_num
    sself_out[...] = jnp.sum(self_num).reshape(1, 1)


_LOG2E = 1.4426950408889634
_LN2 = 0.6931471805599453
_EXP_C = (1.0, 1.0, 0.5, 1.0 / 6, 1.0 / 24, 1.0 / 120, 1.0 / 720, 1.0 / 5040)


def _exp16(l):
    t = l * _LOG2E
    ti = t.astype(jnp.int32)
    k = jnp.where(ti.astype(jnp.float32) > t, ti - 1, ti)
    x = (t - k.astype(jnp.float32)) * _LN2
    p = _EXP_C[7]
    for c in _EXP_C[6::-1]:
        p = p * x + c
    kc = jnp.clip(k, -126, 127)
    two_k = plsc.bitcast(jnp.left_shift(kc + 127, 23), jnp.float32)
    return p * two_k


def _edge_body(src_ids, dst_ids, rs_flat, os_flat, ops_flat, zeros_n,
               acc_out, wsum_out,
               rs_v, os_v, opsc_v, acc_v, src_v, dst_v, wsum_v):
    c = lax.axis_index("c")
    s = lax.axis_index("s")
    wid = s * NC + c
    eg = wid % EDGE_SPLIT
    cg = wid // EDGE_SPLIT
    ebase = eg * E_PER_W

    pltpu.sync_copy(rs_flat, rs_v)
    pltpu.sync_copy(os_flat, os_v)
    pltpu.sync_copy(ops_flat.at[pl.ds(cg * N, N)], opsc_v)
    pltpu.sync_copy(zeros_n, acc_v)

    iota = lax.iota(jnp.int32, 16)

    wsum = jnp.zeros((16,), jnp.float32)
    for ci in range(N_CHUNKS):
        eoff = ebase + ci * CHUNK
        pltpu.sync_copy(src_ids.at[pl.ds(eoff, CHUNK)], src_v)
        pltpu.sync_copy(dst_ids.at[pl.ds(eoff, CHUNK)], dst_v)

        def step(i, wacc):
            idx = i * 16 + iota
            s16 = plsc.load_gather(src_v, [idx])
            d16 = plsc.load_gather(dst_v, [idx])
            l = plsc.load_gather(rs_v, [d16]) + plsc.load_gather(os_v, [s16])
            l = jnp.where(l > 0, l, l * 0.2)
            w = _exp16(l)
            contrib = w * plsc.load_gather(opsc_v, [s16])
            plsc.addupdate_scatter(acc_v, [d16], contrib)
            return wacc + w

        wsum = lax.fori_loop(0, STEPS, step, wsum)

    wsum_v[...] = wsum * jnp.where(cg == 0, 1.0, 0.0)
    pltpu.sync_copy(acc_v, acc_out.at[pl.ds(eg * NF + cg * N, N)])
    pltpu.sync_copy(wsum_v, wsum_out.at[pl.ds(wid * 16, 16)])


def _final_body(p2, acc2, wsum2, sself, out2):
    z = sself[...] + jnp.sum(wsum2[...]).reshape(1, 1)
    acc = acc2[0]
    for eg in range(1, EDGE_SPLIT):
        acc = acc + acc2[eg]
    x = (p2[...] + acc) / z
    out2[...] = jnp.where(x > 0, x, jnp.exp(x) - 1.0)


@jax.jit
def kernel(resources, operations, requirement_edges, W_res, W_op,
           att_self_coef, att_coef):
    edges = requirement_edges.astype(jnp.int32)

    def bf(x):
        return x.astype(jnp.bfloat16).astype(jnp.float32)

    a_self = bf(att_self_coef[:HID]) + bf(att_self_coef[HID:])
    c_res = bf(att_coef[:HID])
    c_ops = bf(att_coef[HID:])

    f32 = jnp.float32
    p_t, ops_t, rs_row, os_row, sself = pl.pallas_call(
        _dense_body,
        out_shape=[
            jax.ShapeDtypeStruct((HID, N), f32),
            jax.ShapeDtypeStruct((HID, N), f32),
            jax.ShapeDtypeStruct((1, N), f32),
            jax.ShapeDtypeStruct((1, N), f32),
            jax.ShapeDtypeStruct((1, 1), f32),
        ],
    )(resources, operations, W_res, W_op, a_self, c_res, c_ops)

    edge_k = pl.kernel(
        _edge_body,
        out_type=[
            jax.ShapeDtypeStruct((EDGE_SPLIT * NF,), f32),
            jax.ShapeDtypeStruct((NW * 16,), f32),
        ],
        mesh=plsc.VectorSubcoreMesh(core_axis_name="c", subcore_axis_name="s"),
        compiler_params=pltpu.CompilerParams(needs_layout_passes=False),
        scratch_types=[
            pltpu.VMEM((N,), f32),
            pltpu.VMEM((N,), f32),
            pltpu.VMEM((N,), f32),
            pltpu.VMEM((N,), f32),
            pltpu.VMEM((CHUNK,), jnp.int32),
            pltpu.VMEM((CHUNK,), jnp.int32),
            pltpu.VMEM((16,), f32),
        ],
    )
    acc_parts, wsum_parts = edge_k(
        edges[0], edges[1], rs_row.reshape(-1), os_row.reshape(-1),
        ops_t.reshape(-1), jnp.zeros((N,), f32))

    out2 = pl.pallas_call(
        _final_body,
        out_shape=jax.ShapeDtypeStruct((NF // 128, 128), f32),
    )(p_t.reshape(NF // 128, 128), acc_parts.reshape(EDGE_SPLIT, NF // 128, 128),
      wsum_parts.reshape(4, 128), sself)
    return out2.reshape(HID, N).T


# speedup vs baseline: 15.7548x; 1.4164x over previous
import jax
import jax.numpy as jnp
from jax import lax
from jax.experimental import pallas as pl
from jax.experimental.pallas import tpu as pltpu
import jax.experimental.pallas.tpu_sc as plsc

N = 10000
E = 320000
HID = 8

NC = 2
NS = 16
NW = NC * NS
COL_SPLIT = HID
EDGE_SPLIT = NW // COL_SPLIT
E_PER_W = E // EDGE_SPLIT
CHUNK = 8000
N_CHUNKS = E_PER_W // CHUNK
STEPS = CHUNK // 16
UNROLL = 4
NF = N * HID


def _dense_body(res_in, ops_in, wres, wop, a_self, c_res, c_ops,
                p_out, ops_out, rs_out, os_out, sself_out):
    dn = (((1,), (1,)), ((), ()))
    res_t = lax.dot_general(wres[...].astype(jnp.bfloat16),
                            res_in[...].astype(jnp.bfloat16), dn,
                            preferred_element_type=jnp.float32)
    ops_t = lax.dot_general(wop[...].astype(jnp.bfloat16),
                            ops_in[...].astype(jnp.bfloat16), dn,
                            preferred_element_type=jnp.float32)
    ops_out[...] = ops_t
    res_tb = res_t.astype(jnp.bfloat16).astype(jnp.float32)
    ops_tb = ops_t.astype(jnp.bfloat16).astype(jnp.float32)
    rs_out[...] = jnp.sum(res_tb * c_res[...], axis=0, keepdims=True)
    os_out[...] = jnp.sum(ops_tb * c_ops[...], axis=0, keepdims=True)
    sl = jnp.sum(res_tb * a_self[...], axis=0, keepdims=True)
    sl = jnp.where(sl > 0, sl, sl * 0.2)
    self_num = jnp.exp(sl)
    p_out[...] = res_t * self_num
    sself_out[...] = jnp.sum(self_num).reshape(1, 1)


def _edge_body(src_ids, dst_ids, rs_flat, os_flat, ops_flat, zeros_n,
               acc_out, wsum_out,
               rs_v, os_v, opsc_v, acc_v, src_v, dst_v, wsum_v):
    c = lax.axis_index("c")
    s = lax.axis_index("s")
    wid = s * NC + c
    eg = wid % EDGE_SPLIT
    cg = wid // EDGE_SPLIT
    ebase = eg * E_PER_W

    pltpu.sync_copy(rs_flat, rs_v)
    pltpu.sync_copy(os_flat, os_v)
    pltpu.sync_copy(ops_flat.at[pl.ds(cg * N, N)], opsc_v)
    pltpu.sync_copy(zeros_n, acc_v)

    wsum = jnp.zeros((16,), jnp.float32)
    for ci in range(N_CHUNKS):
        eoff = ebase + ci * CHUNK
        pltpu.sync_copy(src_ids.at[pl.ds(eoff, CHUNK)], src_v)
        pltpu.sync_copy(dst_ids.at[pl.ds(eoff, CHUNK)], dst_v)

        def step(i, wacc):
            base = i * (16 * UNROLL)
            ws = []
            for u in range(UNROLL):
                s16 = src_v[pl.ds(base + u * 16, 16)]
                d16 = dst_v[pl.ds(base + u * 16, 16)]
                l = (plsc.load_gather(rs_v, [d16])
                     + plsc.load_gather(os_v, [s16]))
                l = jnp.where(l > 0, l, l * 0.2)
                w = jnp.exp(l)
                contrib = w * plsc.load_gather(opsc_v, [s16])
                plsc.addupdate_scatter(acc_v, [d16], contrib)
                ws.append(w)
            return wacc + ((ws[0] + ws[1]) + (ws[2] + ws[3]))

        wsum = lax.fori_loop(0, STEPS // UNROLL, step, wsum)

    wsum_v[...] = wsum * jnp.where(cg == 0, 1.0, 0.0)
    pltpu.sync_copy(acc_v, acc_out.at[pl.ds(eg * NF + cg * N, N)])
    pltpu.sync_copy(wsum_v, wsum_out.at[pl.ds(wid * 16, 16)])


def _final_body(p2, acc2, wsum2, sself, out2):
    z = sself[...] + jnp.sum(wsum2[...]).reshape(1, 1)
    acc = acc2[0]
    for eg in range(1, EDGE_SPLIT):
        acc = acc + acc2[eg]
    x = (p2[...] + acc) / z
    out2[...] = jnp.where(x > 0, x, jnp.exp(x) - 1.0)


@jax.jit
def kernel(resources, operations, requirement_edges, W_res, W_op,
           att_self_coef, att_coef):
    edges = requirement_edges.astype(jnp.int32)

    def bf(x):
        return x.astype(jnp.bfloat16).astype(jnp.float32)

    a_self = bf(att_self_coef[:HID]) + bf(att_self_coef[HID:])
    c_res = bf(att_coef[:HID])
    c_ops = bf(att_coef[HID:])

    f32 = jnp.float32
    p_t, ops_t, rs_row, os_row, sself = pl.pallas_call(
        _dense_body,
        out_shape=[
            jax.ShapeDtypeStruct((HID, N), f32),
            jax.ShapeDtypeStruct((HID, N), f32),
            jax.ShapeDtypeStruct((1, N), f32),
            jax.ShapeDtypeStruct((1, N), f32),
            jax.ShapeDtypeStruct((1, 1), f32),
        ],
    )(resources, operations, W_res, W_op, a_self, c_res, c_ops)

    edge_k = pl.kernel(
        _edge_body,
        out_type=[
            jax.ShapeDtypeStruct((EDGE_SPLIT * NF,), f32),
            jax.ShapeDtypeStruct((NW * 16,), f32),
        ],
        mesh=plsc.VectorSubcoreMesh(core_axis_name="c", subcore_axis_name="s"),
        compiler_params=pltpu.CompilerParams(needs_layout_passes=False),
        scratch_types=[
            pltpu.VMEM((N,), f32),
            pltpu.VMEM((N,), f32),
            pltpu.VMEM((N,), f32),
            pltpu.VMEM((N,), f32),
            pltpu.VMEM((CHUNK,), jnp.int32),
            pltpu.VMEM((CHUNK,), jnp.int32),
            pltpu.VMEM((16,), f32),
        ],
    )
    acc_parts, wsum_parts = edge_k(
        edges[0], edges[1], rs_row.reshape(-1), os_row.reshape(-1),
        ops_t.reshape(-1), jnp.zeros((N,), f32))

    out2 = pl.pallas_call(
        _final_body,
        out_shape=jax.ShapeDtypeStruct((NF // 128, 128), f32),
    )(p_t.reshape(NF // 128, 128), acc_parts.reshape(EDGE_SPLIT, NF // 128, 128),
      wsum_parts.reshape(4, 128), sself)
    return out2.reshape(HID, N).T


# speedup vs baseline: 20.7058x; 1.3142x over previous
import jax
import jax.numpy as jnp
from jax import lax
from jax.experimental import pallas as pl
from jax.experimental.pallas import tpu as pltpu
import jax.experimental.pallas.tpu_sc as plsc

N = 10000
E = 320000
HID = 8

NC = 2
NS = 16
NW = NC * NS
COL_SPLIT = HID
EDGE_SPLIT = NW // COL_SPLIT
E_PER_W = E // EDGE_SPLIT
CHUNK = 10000
N_CHUNKS = E_PER_W // CHUNK
UNROLL = 5
assert CHUNK % (16 * UNROLL) == 0
NF = N * HID


def _dense_body(res_in, ops_in, wres, wop, a_self, c_res, c_ops,
                p_out, ops_out, rs_out, os_out, sself_out):
    dn = (((1,), (1,)), ((), ()))
    res_t = lax.dot_general(wres[...].astype(jnp.bfloat16),
                            res_in[...].astype(jnp.bfloat16), dn,
                            preferred_element_type=jnp.float32)
    ops_t = lax.dot_general(wop[...].astype(jnp.bfloat16),
                            ops_in[...].astype(jnp.bfloat16), dn,
                            preferred_element_type=jnp.float32)
    ops_out[...] = ops_t
    res_tb = res_t.astype(jnp.bfloat16).astype(jnp.float32)
    ops_tb = ops_t.astype(jnp.bfloat16).astype(jnp.float32)
    rs_out[...] = jnp.sum(res_tb * c_res[...], axis=0, keepdims=True)
    os_out[...] = jnp.sum(ops_tb * c_ops[...], axis=0, keepdims=True)
    sl = jnp.sum(res_tb * a_self[...], axis=0, keepdims=True)
    sl = jnp.where(sl > 0, sl, sl * 0.2)
    self_num = jnp.exp(sl)
    p_out[...] = res_t * self_num
    sself_out[...] = jnp.sum(self_num).reshape(1, 1)


def _edge_body(src_ids, dst_ids, rs_flat, os_flat, ops_flat, zeros_n,
               acc_out, wsum_out,
               rs_v, os_v, opsc_v, acc_v, src_v, dst_v, w_v, wsum_v, w_sp):
    c = lax.axis_index("c")
    s = lax.axis_index("s")
    eg = 2 * c + (s & 1)
    cg = s >> 1
    wid = eg * COL_SPLIT + cg
    ebase = eg * E_PER_W
    sp_base = (eg & 1) * E_PER_W

    pltpu.sync_copy(rs_flat, rs_v)
    pltpu.sync_copy(os_flat, os_v)
    pltpu.sync_copy(ops_flat.at[pl.ds(cg * N, N)], opsc_v)
    pltpu.sync_copy(zeros_n, acc_v)

    p1off = ebase + cg * CHUNK
    pltpu.sync_copy(src_ids.at[pl.ds(p1off, CHUNK)], src_v)
    pltpu.sync_copy(dst_ids.at[pl.ds(p1off, CHUNK)], dst_v)

    def stepw(i, wacc):
        base = i * (16 * UNROLL)
        ws = []
        for u in range(UNROLL):
            s16 = src_v[pl.ds(base + u * 16, 16)]
            d16 = dst_v[pl.ds(base + u * 16, 16)]
            l = (plsc.load_gather(rs_v, [d16])
                 + plsc.load_gather(os_v, [s16]))
            l = jnp.where(l > 0, l, l * 0.2)
            w = jnp.exp(l)
            w_v[pl.ds(base + u * 16, 16)] = w
            ws.append(w)
        return wacc + (((ws[0] + ws[1]) + (ws[2] + ws[3])) + ws[4])

    wsum = lax.fori_loop(0, CHUNK // (16 * UNROLL), stepw,
                         jnp.zeros((16,), jnp.float32))
    wsum_v[...] = wsum
    pltpu.sync_copy(w_v, w_sp.at[pl.ds(sp_base + cg * CHUNK, CHUNK)])
    plsc.subcore_barrier()

    for ci in range(N_CHUNKS):
        coff = ebase + ci * CHUNK
        pltpu.sync_copy(src_ids.at[pl.ds(coff, CHUNK)], src_v)
        pltpu.sync_copy(dst_ids.at[pl.ds(coff, CHUNK)], dst_v)
        pltpu.sync_copy(w_sp.at[pl.ds(sp_base + ci * CHUNK, CHUNK)], w_v)

        def step2(i, carry):
            base = i * (16 * UNROLL)
            for u in range(UNROLL):
                s16 = src_v[pl.ds(base + u * 16, 16)]
                d16 = dst_v[pl.ds(base + u * 16, 16)]
                w16 = w_v[pl.ds(base + u * 16, 16)]
                contrib = w16 * plsc.load_gather(opsc_v, [s16])
                plsc.addupdate_scatter(acc_v, [d16], contrib)
            return carry

        lax.fori_loop(0, CHUNK // (16 * UNROLL), step2, 0)

    pltpu.sync_copy(acc_v, acc_out.at[pl.ds(eg * NF + cg * N, N)])
    pltpu.sync_copy(wsum_v, wsum_out.at[pl.ds(wid * 16, 16)])


def _final_body(p2, acc2, wsum2, sself, out2):
    z = sself[...] + jnp.sum(wsum2[...]).reshape(1, 1)
    acc = acc2[0]
    for eg in range(1, EDGE_SPLIT):
        acc = acc + acc2[eg]
    x = (p2[...] + acc) / z
    out2[...] = jnp.where(x > 0, x, jnp.exp(x) - 1.0)


@jax.jit
def kernel(resources, operations, requirement_edges, W_res, W_op,
           att_self_coef, att_coef):
    edges = requirement_edges.astype(jnp.int32)

    def bf(x):
        return x.astype(jnp.bfloat16).astype(jnp.float32)

    a_self = bf(att_self_coef[:HID]) + bf(att_self_coef[HID:])
    c_res = bf(att_coef[:HID])
    c_ops = bf(att_coef[HID:])

    f32 = jnp.float32
    p_t, ops_t, rs_row, os_row, sself = pl.pallas_call(
        _dense_body,
        out_shape=[
            jax.ShapeDtypeStruct((HID, N), f32),
            jax.ShapeDtypeStruct((HID, N), f32),
            jax.ShapeDtypeStruct((1, N), f32),
            jax.ShapeDtypeStruct((1, N), f32),
            jax.ShapeDtypeStruct((1, 1), f32),
        ],
    )(resources, operations, W_res, W_op, a_self, c_res, c_ops)

    edge_k = pl.kernel(
        _edge_body,
        out_type=[
            jax.ShapeDtypeStruct((EDGE_SPLIT * NF,), f32),
            jax.ShapeDtypeStruct((NW * 16,), f32),
        ],
        mesh=plsc.VectorSubcoreMesh(core_axis_name="c", subcore_axis_name="s"),
        compiler_params=pltpu.CompilerParams(needs_layout_passes=False),
        scratch_types=[
            pltpu.VMEM((N,), f32),
            pltpu.VMEM((N,), f32),
            pltpu.VMEM((N,), f32),
            pltpu.VMEM((N,), f32),
            pltpu.VMEM((CHUNK,), jnp.int32),
            pltpu.VMEM((CHUNK,), jnp.int32),
            pltpu.VMEM((CHUNK,), f32),
            pltpu.VMEM((16,), f32),
            pltpu.VMEM_SHARED((2 * E_PER_W,), f32),
        ],
    )
    acc_parts, wsum_parts = edge_k(
        edges[0], edges[1], rs_row.reshape(-1), os_row.reshape(-1),
        ops_t.reshape(-1), jnp.zeros((N,), f32))

    out2 = pl.pallas_call(
        _final_body,
        out_shape=jax.ShapeDtypeStruct((NF // 128, 128), f32),
    )(p_t.reshape(NF // 128, 128), acc_parts.reshape(EDGE_SPLIT, NF // 128, 128),
      wsum_parts.reshape(4, 128), sself)
    return out2.reshape(HID, N).T
